# baseline (speedup 1.0000x reference)
import dataclasses

import jax
import jax.numpy as jnp
from jax import lax
from jax.experimental import pallas as pl
from jax.experimental.pallas import tpu as pltpu
from jax.experimental.pallas import tpu_sc as plsc

EMB_DIM = 64
WIDE = 128
NUM_WORKERS = 32
NB = 8
TC_BLOCK = 512


def _transpose_block(tt_ref, w_ref):
    w_ref[:, :EMB_DIM] = jnp.transpose(tt_ref[...], (1, 0))


def _widen_table(table, n_rows):
    tt = jnp.transpose(table)
    grid = (n_rows + TC_BLOCK - 1) // TC_BLOCK
    return pl.pallas_call(
        _transpose_block,
        grid=(grid,),
        in_specs=[pl.BlockSpec((EMB_DIM, TC_BLOCK), lambda i: (0, i))],
        out_specs=pl.BlockSpec((TC_BLOCK, WIDE), lambda i: (i, 0)),
        out_shape=jax.ShapeDtypeStruct((grid * TC_BLOCK, WIDE), table.dtype),
        compiler_params=pltpu.CompilerParams(
            dimension_semantics=("arbitrary",),
        ),
    )(tt)


def kernel(x, table):
    batch, hist = x.shape
    per_worker = batch // NUM_WORKERS
    n_chunks = per_worker // NB
    assert per_worker * NUM_WORKERS == batch and n_chunks * NB == per_worker
    assert n_chunks % 2 == 0

    n_rows = table.shape[0] - 1
    table_w = _widen_table(table, n_rows)

    mesh = plsc.VectorSubcoreMesh(core_axis_name="c", subcore_axis_name="s")
    cp = dataclasses.replace(pltpu.CompilerParams(), use_tc_tiling_on_sc=False)

    @pl.kernel(
        out_type=jax.ShapeDtypeStruct((batch, hist, EMB_DIM), table.dtype),
        mesh=mesh,
        scratch_types=[
            pltpu.VMEM((NB, hist), jnp.int32),
            pltpu.VMEM((NB, hist), jnp.int32),
            pltpu.VMEM((NB, hist, WIDE), jnp.float32),
            pltpu.VMEM((NB, hist, WIDE), jnp.float32),
            pltpu.SemaphoreType.DMA,
            pltpu.SemaphoreType.DMA,
            pltpu.SemaphoreType.DMA,
            pltpu.SemaphoreType.DMA,
        ],
        compiler_params=cp,
    )
    def gather_kernel(
        x_hbm, table_hbm, out_hbm, idx0, idx1, sv0, sv1, semg0, semg1, semo0, semo1
    ):
        wid = lax.axis_index("s") * 2 + lax.axis_index("c")
        b0 = wid * per_worker

        def fire(chunk, idx_v, s_v, semg):
            pltpu.sync_copy(x_hbm.at[pl.ds(b0 + chunk * NB, NB)], idx_v)
            for j in range(NB):
                pltpu.async_copy(table_hbm.at[idx_v.at[j]], s_v.at[j], semg)

        def drain(idx_v, s_v, semg):
            for j in range(NB):
                pltpu.make_async_copy(table_hbm.at[idx_v.at[j]], s_v.at[j], semg).wait()

        def store(chunk, s_v, semo):
            return pltpu.async_copy(
                s_v.at[:, :, pl.ds(0, EMB_DIM)],
                out_hbm.at[pl.ds(b0 + chunk * NB, NB)],
                semo,
            )

        def store_wait(chunk, s_v, semo):
            pltpu.make_async_copy(
                s_v.at[:, :, pl.ds(0, EMB_DIM)],
                out_hbm.at[pl.ds(b0 + chunk * NB, NB)],
                semo,
            ).wait()

        fire(0, idx0, sv0, semg0)
        fire(1, idx1, sv1, semg1)

        @pl.loop(0, n_chunks // 2 - 1)
        def _(i):
            ca = 2 * i
            cb = ca + 1
            drain(idx0, sv0, semg0)
            store(ca, sv0, semo0)
            drain(idx1, sv1, semg1)
            store(cb, sv1, semo1)
            store_wait(ca, sv0, semo0)
            fire(ca + 2, idx0, sv0, semg0)
            store_wait(cb, sv1, semo1)
            fire(cb + 2, idx1, sv1, semg1)

        drain(idx0, sv0, semg0)
        store(n_chunks - 2, sv0, semo0)
        drain(idx1, sv1, semg1)
        store(n_chunks - 1, sv1, semo1)
        store_wait(n_chunks - 2, sv0, semo0)
        store_wait(n_chunks - 1, sv1, semo1)

    return gather_kernel(x, table_w)


# speedup vs baseline: 1.8489x; 1.8489x over previous
import dataclasses

import jax
import jax.numpy as jnp
from jax import lax
from jax.experimental import pallas as pl
from jax.experimental.pallas import tpu as pltpu
from jax.experimental.pallas import tpu_sc as plsc

EMB_DIM = 64
WIDE = 128
NUM_WORKERS = 32
NB = 8
TC_BLOCK = 8192


def _transpose_block(tt_ref, w_ref):
    w_ref[:, :EMB_DIM] = jnp.transpose(tt_ref[...], (1, 0))


def _widen_table(table, n_rows):
    tt = jnp.transpose(table)
    grid = (n_rows + TC_BLOCK - 1) // TC_BLOCK
    return pl.pallas_call(
        _transpose_block,
        grid=(grid,),
        in_specs=[pl.BlockSpec((EMB_DIM, TC_BLOCK), lambda i: (0, i))],
        out_specs=pl.BlockSpec((TC_BLOCK, WIDE), lambda i: (i, 0)),
        out_shape=jax.ShapeDtypeStruct((grid * TC_BLOCK, WIDE), table.dtype),
        compiler_params=pltpu.CompilerParams(
            dimension_semantics=("arbitrary",),
        ),
    )(tt)


def kernel(x, table):
    batch, hist = x.shape
    per_worker = batch // NUM_WORKERS
    n_chunks = per_worker // NB
    assert per_worker * NUM_WORKERS == batch and n_chunks * NB == per_worker
    assert n_chunks % 2 == 0

    n_rows = table.shape[0] - 1
    table_w = _widen_table(table, n_rows)

    mesh = plsc.VectorSubcoreMesh(core_axis_name="c", subcore_axis_name="s")
    cp = dataclasses.replace(pltpu.CompilerParams(), use_tc_tiling_on_sc=False)

    @pl.kernel(
        out_type=jax.ShapeDtypeStruct((batch, hist, EMB_DIM), table.dtype),
        mesh=mesh,
        scratch_types=[
            pltpu.VMEM((NB, hist), jnp.int32),
            pltpu.VMEM((NB, hist), jnp.int32),
            pltpu.VMEM((NB, hist, WIDE), jnp.float32),
            pltpu.VMEM((NB, hist, WIDE), jnp.float32),
            pltpu.SemaphoreType.DMA,
            pltpu.SemaphoreType.DMA,
            pltpu.SemaphoreType.DMA,
            pltpu.SemaphoreType.DMA,
        ],
        compiler_params=cp,
    )
    def gather_kernel(
        x_hbm, table_hbm, out_hbm, idx0, idx1, sv0, sv1, semg0, semg1, semo0, semo1
    ):
        wid = lax.axis_index("s") * 2 + lax.axis_index("c")
        b0 = wid * per_worker

        def fire(chunk, idx_v, s_v, semg):
            pltpu.sync_copy(x_hbm.at[pl.ds(b0 + chunk * NB, NB)], idx_v)
            for j in range(NB):
                pltpu.async_copy(table_hbm.at[idx_v.at[j]], s_v.at[j], semg)

        def drain(idx_v, s_v, semg):
            for j in range(NB):
                pltpu.make_async_copy(table_hbm.at[idx_v.at[j]], s_v.at[j], semg).wait()

        def store(chunk, s_v, semo):
            return pltpu.async_copy(
                s_v.at[:, :, pl.ds(0, EMB_DIM)],
                out_hbm.at[pl.ds(b0 + chunk * NB, NB)],
                semo,
            )

        def store_wait(chunk, s_v, semo):
            pltpu.make_async_copy(
                s_v.at[:, :, pl.ds(0, EMB_DIM)],
                out_hbm.at[pl.ds(b0 + chunk * NB, NB)],
                semo,
            ).wait()

        fire(0, idx0, sv0, semg0)
        fire(1, idx1, sv1, semg1)

        @pl.loop(0, n_chunks // 2 - 1)
        def _(i):
            ca = 2 * i
            cb = ca + 1
            drain(idx0, sv0, semg0)
            store(ca, sv0, semo0)
            drain(idx1, sv1, semg1)
            store(cb, sv1, semo1)
            store_wait(ca, sv0, semo0)
            fire(ca + 2, idx0, sv0, semg0)
            store_wait(cb, sv1, semo1)
            fire(cb + 2, idx1, sv1, semg1)

        drain(idx0, sv0, semg0)
        store(n_chunks - 2, sv0, semo0)
        drain(idx1, sv1, semg1)
        store(n_chunks - 1, sv1, semo1)
        store_wait(n_chunks - 2, sv0, semo0)
        store_wait(n_chunks - 1, sv1, semo1)

    return gather_kernel(x, table_w)


# speedup vs baseline: 1.8773x; 1.0153x over previous
import dataclasses

import jax
import jax.numpy as jnp
from jax import lax
from jax.experimental import pallas as pl
from jax.experimental.pallas import tpu as pltpu
from jax.experimental.pallas import tpu_sc as plsc

EMB_DIM = 64
WIDE = 128
NUM_WORKERS = 32
NB = 8
TC_BLOCK = 16384


def _transpose_block(tt_ref, w_ref):
    w_ref[:, :EMB_DIM] = jnp.transpose(tt_ref[...], (1, 0))


def _widen_table(table, n_rows):
    tt = jnp.transpose(table)
    grid = (n_rows + TC_BLOCK - 1) // TC_BLOCK
    return pl.pallas_call(
        _transpose_block,
        grid=(grid,),
        in_specs=[pl.BlockSpec((EMB_DIM, TC_BLOCK), lambda i: (0, i))],
        out_specs=pl.BlockSpec((TC_BLOCK, WIDE), lambda i: (i, 0)),
        out_shape=jax.ShapeDtypeStruct((grid * TC_BLOCK, WIDE), table.dtype),
        compiler_params=pltpu.CompilerParams(
            dimension_semantics=("arbitrary",),
        ),
    )(tt)


def kernel(x, table):
    batch, hist = x.shape
    per_worker = batch // NUM_WORKERS
    n_chunks = per_worker // NB
    assert per_worker * NUM_WORKERS == batch and n_chunks * NB == per_worker
    assert n_chunks % 2 == 0

    n_rows = table.shape[0] - 1
    table_w = _widen_table(table, n_rows)

    mesh = plsc.VectorSubcoreMesh(core_axis_name="c", subcore_axis_name="s")
    cp = dataclasses.replace(pltpu.CompilerParams(), use_tc_tiling_on_sc=False)

    @pl.kernel(
        out_type=jax.ShapeDtypeStruct((batch, hist, EMB_DIM), table.dtype),
        mesh=mesh,
        scratch_types=[
            pltpu.VMEM((NB, hist), jnp.int32),
            pltpu.VMEM((NB, hist), jnp.int32),
            pltpu.VMEM((NB, hist, WIDE), jnp.float32),
            pltpu.VMEM((NB, hist, WIDE), jnp.float32),
            pltpu.SemaphoreType.DMA,
            pltpu.SemaphoreType.DMA,
            pltpu.SemaphoreType.DMA,
            pltpu.SemaphoreType.DMA,
        ],
        compiler_params=cp,
    )
    def gather_kernel(
        x_hbm, table_hbm, out_hbm, idx0, idx1, sv0, sv1, semg0, semg1, semo0, semo1
    ):
        wid = lax.axis_index("s") * 2 + lax.axis_index("c")
        b0 = wid * per_worker

        def fire(chunk, idx_v, s_v, semg):
            pltpu.sync_copy(x_hbm.at[pl.ds(b0 + chunk * NB, NB)], idx_v)
            for j in range(NB):
                pltpu.async_copy(table_hbm.at[idx_v.at[j]], s_v.at[j], semg)

        def drain(idx_v, s_v, semg):
            for j in range(NB):
                pltpu.make_async_copy(table_hbm.at[idx_v.at[j]], s_v.at[j], semg).wait()

        def store(chunk, s_v, semo):
            return pltpu.async_copy(
                s_v.at[:, :, pl.ds(0, EMB_DIM)],
                out_hbm.at[pl.ds(b0 + chunk * NB, NB)],
                semo,
            )

        def store_wait(chunk, s_v, semo):
            pltpu.make_async_copy(
                s_v.at[:, :, pl.ds(0, EMB_DIM)],
                out_hbm.at[pl.ds(b0 + chunk * NB, NB)],
                semo,
            ).wait()

        fire(0, idx0, sv0, semg0)
        fire(1, idx1, sv1, semg1)

        @pl.loop(0, n_chunks // 2 - 1)
        def _(i):
            ca = 2 * i
            cb = ca + 1
            drain(idx0, sv0, semg0)
            store(ca, sv0, semo0)
            drain(idx1, sv1, semg1)
            store(cb, sv1, semo1)
            store_wait(ca, sv0, semo0)
            fire(ca + 2, idx0, sv0, semg0)
            store_wait(cb, sv1, semo1)
            fire(cb + 2, idx1, sv1, semg1)

        drain(idx0, sv0, semg0)
        store(n_chunks - 2, sv0, semo0)
        drain(idx1, sv1, semg1)
        store(n_chunks - 1, sv1, semo1)
        store_wait(n_chunks - 2, sv0, semo0)
        store_wait(n_chunks - 1, sv1, semo1)

    return gather_kernel(x, table_w)


# speedup vs baseline: 1.8862x; 1.0048x over previous
import dataclasses

import jax
import jax.numpy as jnp
from jax import lax
from jax.experimental import pallas as pl
from jax.experimental.pallas import tpu as pltpu
from jax.experimental.pallas import tpu_sc as plsc

EMB_DIM = 64
WIDE = 128
NUM_WORKERS = 32
NB = 8
TC_BLOCK = 32768


def _transpose_block(tt_ref, w_ref):
    w_ref[:, :EMB_DIM] = jnp.transpose(tt_ref[...], (1, 0))


def _widen_table(table, n_rows):
    tt = jnp.transpose(table)
    grid = (n_rows + TC_BLOCK - 1) // TC_BLOCK
    return pl.pallas_call(
        _transpose_block,
        grid=(grid,),
        in_specs=[pl.BlockSpec((EMB_DIM, TC_BLOCK), lambda i: (0, i))],
        out_specs=pl.BlockSpec((TC_BLOCK, WIDE), lambda i: (i, 0)),
        out_shape=jax.ShapeDtypeStruct((grid * TC_BLOCK, WIDE), table.dtype),
        compiler_params=pltpu.CompilerParams(
            dimension_semantics=("arbitrary",),
        ),
    )(tt)


def kernel(x, table):
    batch, hist = x.shape
    per_worker = batch // NUM_WORKERS
    n_chunks = per_worker // NB
    assert per_worker * NUM_WORKERS == batch and n_chunks * NB == per_worker
    assert n_chunks % 2 == 0

    n_rows = table.shape[0] - 1
    table_w = _widen_table(table, n_rows)

    mesh = plsc.VectorSubcoreMesh(core_axis_name="c", subcore_axis_name="s")
    cp = dataclasses.replace(pltpu.CompilerParams(), use_tc_tiling_on_sc=False)

    @pl.kernel(
        out_type=jax.ShapeDtypeStruct((batch, hist, EMB_DIM), table.dtype),
        mesh=mesh,
        scratch_types=[
            pltpu.VMEM((NB, hist), jnp.int32),
            pltpu.VMEM((NB, hist), jnp.int32),
            pltpu.VMEM((NB, hist, WIDE), jnp.float32),
            pltpu.VMEM((NB, hist, WIDE), jnp.float32),
            pltpu.SemaphoreType.DMA,
            pltpu.SemaphoreType.DMA,
            pltpu.SemaphoreType.DMA,
            pltpu.SemaphoreType.DMA,
        ],
        compiler_params=cp,
    )
    def gather_kernel(
        x_hbm, table_hbm, out_hbm, idx0, idx1, sv0, sv1, semg0, semg1, semo0, semo1
    ):
        wid = lax.axis_index("s") * 2 + lax.axis_index("c")
        b0 = wid * per_worker

        def fire(chunk, idx_v, s_v, semg):
            pltpu.sync_copy(x_hbm.at[pl.ds(b0 + chunk * NB, NB)], idx_v)
            for j in range(NB):
                pltpu.async_copy(table_hbm.at[idx_v.at[j]], s_v.at[j], semg)

        def drain(idx_v, s_v, semg):
            for j in range(NB):
                pltpu.make_async_copy(table_hbm.at[idx_v.at[j]], s_v.at[j], semg).wait()

        def store(chunk, s_v, semo):
            return pltpu.async_copy(
                s_v.at[:, :, pl.ds(0, EMB_DIM)],
                out_hbm.at[pl.ds(b0 + chunk * NB, NB)],
                semo,
            )

        def store_wait(chunk, s_v, semo):
            pltpu.make_async_copy(
                s_v.at[:, :, pl.ds(0, EMB_DIM)],
                out_hbm.at[pl.ds(b0 + chunk * NB, NB)],
                semo,
            ).wait()

        fire(0, idx0, sv0, semg0)
        fire(1, idx1, sv1, semg1)

        @pl.loop(0, n_chunks // 2 - 1)
        def _(i):
            ca = 2 * i
            cb = ca + 1
            drain(idx0, sv0, semg0)
            store(ca, sv0, semo0)
            drain(idx1, sv1, semg1)
            store(cb, sv1, semo1)
            store_wait(ca, sv0, semo0)
            fire(ca + 2, idx0, sv0, semg0)
            store_wait(cb, sv1, semo1)
            fire(cb + 2, idx1, sv1, semg1)

        drain(idx0, sv0, semg0)
        store(n_chunks - 2, sv0, semo0)
        drain(idx1, sv1, semg1)
        store(n_chunks - 1, sv1, semo1)
        store_wait(n_chunks - 2, sv0, semo0)
        store_wait(n_chunks - 1, sv1, semo1)

    return gather_kernel(x, table_w)


# speedup vs baseline: 2.7159x; 1.4399x over previous
import dataclasses

import jax
import jax.numpy as jnp
from jax import lax
from jax.experimental import pallas as pl
from jax.experimental.pallas import tpu as pltpu
from jax.experimental.pallas import tpu_sc as plsc

EMB_DIM = 64
WIDE = 128
NUM_WORKERS = 32
NB = 8
TC_BLOCK = 32768


def _transpose_block(tt_ref, w_ref):
    w_ref[:, :EMB_DIM] = jnp.transpose(tt_ref[...], (1, 0))


def _widen_table(table, n_rows):
    tt = jnp.transpose(table)
    grid = (n_rows + TC_BLOCK - 1) // TC_BLOCK
    return pl.pallas_call(
        _transpose_block,
        grid=(grid,),
        in_specs=[pl.BlockSpec((EMB_DIM, TC_BLOCK), lambda i: (0, i))],
        out_specs=pl.BlockSpec((TC_BLOCK, WIDE), lambda i: (i, 0)),
        out_shape=jax.ShapeDtypeStruct((grid * TC_BLOCK, WIDE), table.dtype),
        compiler_params=pltpu.CompilerParams(
            dimension_semantics=("arbitrary",),
        ),
    )(tt)


def kernel(x, table):
    batch, hist = x.shape
    per_worker = batch // NUM_WORKERS
    n_chunks = per_worker // NB
    assert per_worker * NUM_WORKERS == batch and n_chunks * NB == per_worker
    assert n_chunks % 2 == 0

    n_rows = table.shape[0] - 1
    table_w = _widen_table(table, n_rows)

    mesh = plsc.VectorSubcoreMesh(core_axis_name="c", subcore_axis_name="s")
    cp = dataclasses.replace(pltpu.CompilerParams(), use_tc_tiling_on_sc=False)

    hist_pad = 56

    @pl.kernel(
        out_type=jax.ShapeDtypeStruct((batch, hist_pad, WIDE), table.dtype),
        mesh=mesh,
        scratch_types=[
            pltpu.VMEM((NB, hist), jnp.int32),
            pltpu.VMEM((NB, hist), jnp.int32),
            pltpu.VMEM((NB, hist, WIDE), jnp.float32),
            pltpu.VMEM((NB, hist, WIDE), jnp.float32),
            pltpu.SemaphoreType.DMA,
            pltpu.SemaphoreType.DMA,
            pltpu.SemaphoreType.DMA,
            pltpu.SemaphoreType.DMA,
        ],
        compiler_params=cp,
    )
    def gather_kernel(
        x_hbm, table_hbm, out_hbm, idx0, idx1, sv0, sv1, semg0, semg1, semo0, semo1
    ):
        wid = lax.axis_index("s") * 2 + lax.axis_index("c")
        b0 = wid * per_worker

        def fire(chunk, idx_v, s_v, semg):
            pltpu.sync_copy(x_hbm.at[pl.ds(b0 + chunk * NB, NB)], idx_v)
            for j in range(NB):
                pltpu.async_copy(table_hbm.at[idx_v.at[j]], s_v.at[j], semg)

        def drain(idx_v, s_v, semg):
            for j in range(NB):
                pltpu.make_async_copy(table_hbm.at[idx_v.at[j]], s_v.at[j], semg).wait()

        def store(chunk, s_v, semo):
            return pltpu.async_copy(
                s_v,
                out_hbm.at[pl.ds(b0 + chunk * NB, NB), pl.ds(0, hist)],
                semo,
            )

        def store_wait(chunk, s_v, semo):
            pltpu.make_async_copy(
                s_v,
                out_hbm.at[pl.ds(b0 + chunk * NB, NB), pl.ds(0, hist)],
                semo,
            ).wait()

        fire(0, idx0, sv0, semg0)
        fire(1, idx1, sv1, semg1)

        @pl.loop(0, n_chunks // 2 - 1)
        def _(i):
            ca = 2 * i
            cb = ca + 1
            drain(idx0, sv0, semg0)
            store(ca, sv0, semo0)
            drain(idx1, sv1, semg1)
            store(cb, sv1, semo1)
            store_wait(ca, sv0, semo0)
            fire(ca + 2, idx0, sv0, semg0)
            store_wait(cb, sv1, semo1)
            fire(cb + 2, idx1, sv1, semg1)

        drain(idx0, sv0, semg0)
        store(n_chunks - 2, sv0, semo0)
        drain(idx1, sv1, semg1)
        store(n_chunks - 1, sv1, semo1)
        store_wait(n_chunks - 2, sv0, semo0)
        store_wait(n_chunks - 1, sv1, semo1)

    out = gather_kernel(x, table_w)
    return out[:, :hist, :EMB_DIM]
